# baseline (speedup 1.0000x reference)
import jax
import jax.numpy as jnp
from jax.experimental import pallas as pl


def _make_body(S, D):
    def body(x_ref, pe_ref, out_ref):
        pe = pe_ref[...]
        for s in range(S):
            sl = pl.ds(s * D, D)
            out_ref[0, :, sl] = x_ref[0, :, sl] + pe
    return body


def kernel(x, pos_embedding):
    B, T, S, D = x.shape
    TB = 512
    x3 = x.reshape(B, T, S * D)
    grid = (T // TB, B)
    out = pl.pallas_call(
        _make_body(S, D),
        grid=grid,
        in_specs=[
            pl.BlockSpec((1, TB, S * D), lambda t, b: (b, t, 0)),
            pl.BlockSpec((TB, D), lambda t, b: (t, 0)),
        ],
        out_specs=pl.BlockSpec((1, TB, S * D), lambda t, b: (b, t, 0)),
        out_shape=jax.ShapeDtypeStruct((B, T, S * D), x.dtype),
    )(x3, pos_embedding)
    return out.reshape(B, T, S, D)


# speedup vs baseline: 4.6737x; 4.6737x over previous
import jax
import jax.numpy as jnp
from jax.experimental import pallas as pl


def _make_body(S):
    def body(x_ref, pe_ref, out_ref):
        pe = pe_ref[...]
        for s in range(S):
            out_ref[0, :, s, :] = x_ref[0, :, s, :] + pe
    return body


def kernel(x, pos_embedding):
    B, T, S, D = x.shape
    TB = 512
    grid = (T // TB, B)
    return pl.pallas_call(
        _make_body(S),
        grid=grid,
        in_specs=[
            pl.BlockSpec((1, TB, S, D), lambda t, b: (b, t, 0, 0)),
            pl.BlockSpec((TB, D), lambda t, b: (t, 0)),
        ],
        out_specs=pl.BlockSpec((1, TB, S, D), lambda t, b: (b, t, 0, 0)),
        out_shape=jax.ShapeDtypeStruct((B, T, S, D), x.dtype),
    )(x, pos_embedding)
